# Initial kernel scaffold; baseline (speedup 1.0000x reference)
import functools

import jax
import jax.numpy as jnp
from jax import lax
from jax.experimental import pallas as pl
from jax.experimental.pallas import tpu as pltpu
from jax.experimental.pallas import tpu_sc as plsc

EMBED = 2048
NE = 16
NTOK = 16384
ACTIVE = 2

NC, NS, L = 2, 16, 16
NW = NC * NS
TPW = NTOK // NW
GROUPS = TPW // L

ROW_BLK = 1024


def _score_body(x_ref, w_ref, b_ref, o_ref):
    acc = jnp.dot(x_ref[...], w_ref[...], preferred_element_type=jnp.float32)
    o_ref[...] = jnp.maximum(acc + b_ref[...], 0.0)


def _scores_tc(x, W, b2):
    return pl.pallas_call(
        _score_body,
        grid=(NTOK // ROW_BLK,),
        in_specs=[
            pl.BlockSpec((ROW_BLK, EMBED), lambda i: (i, 0)),
            pl.BlockSpec((EMBED, NE), lambda i: (0, 0)),
            pl.BlockSpec((1, NE), lambda i: (0, 0)),
        ],
        out_specs=pl.BlockSpec((ROW_BLK, NE), lambda i: (i, 0)),
        out_shape=jax.ShapeDtypeStruct((NTOK, NE), jnp.float32),
    )(x, W, b2)


@functools.partial(
    pl.kernel,
    out_type=(
        jax.ShapeDtypeStruct((NTOK, NE), jnp.float32),
        jax.ShapeDtypeStruct((NTOK, NE), jnp.float32),
    ),
    mesh=plsc.VectorSubcoreMesh(core_axis_name="c", subcore_axis_name="s"),
    scratch_types=[
        pltpu.VMEM((TPW, NE), jnp.float32),
        pltpu.VMEM((TPW, NE), jnp.float32),
        pltpu.VMEM((TPW, NE), jnp.float32),
    ],
)
def _route_sc(scores_hbm, rw_hbm, mk_hbm, s_v, rw_v, mk_v):
    wid = lax.axis_index("s") * NC + lax.axis_index("c")
    base = wid * TPW
    pltpu.sync_copy(scores_hbm.at[pl.ds(base, TPW)], s_v)

    zeros = jnp.zeros((L,), jnp.float32)
    ones = jnp.ones((L,), jnp.float32)

    def group(g, carry):
        r0 = g * L
        rows = r0 + lax.iota(jnp.int32, L)
        m1 = jnp.full((L,), -jnp.inf, jnp.float32)
        m2 = jnp.full((L,), -jnp.inf, jnp.float32)
        i1 = jnp.zeros((L,), jnp.int32)
        i2 = jnp.zeros((L,), jnp.int32)
        for e in range(NE):
            col = jnp.full((L,), e, jnp.int32)
            v = plsc.load_gather(s_v, [rows, col])
            gt1 = v > m1
            gt2 = jnp.logical_and(jnp.logical_not(gt1), v > m2)
            i2 = jnp.where(gt1, i1, jnp.where(gt2, e, i2))
            m2 = jnp.where(gt1, m1, jnp.where(gt2, v, m2))
            i1 = jnp.where(gt1, e, i1)
            m1 = jnp.where(gt1, v, m1)
        t = jnp.exp(m2 - m1)
        den = 1.0 + t
        w1 = 1.0 / den
        w2 = t / den
        for r in range(L):
            rw_v[r0 + r, :] = zeros
            mk_v[r0 + r, :] = zeros
        plsc.store_scatter(mk_v, [rows, i1], ones)
        plsc.store_scatter(mk_v, [rows, i2], ones)
        plsc.store_scatter(rw_v, [rows, i1], w1)
        plsc.store_scatter(rw_v, [rows, i2], w2)
        return carry

    lax.fori_loop(0, GROUPS, group, 0)
    pltpu.sync_copy(rw_v, rw_hbm.at[pl.ds(base, TPW)])
    pltpu.sync_copy(mk_v, mk_hbm.at[pl.ds(base, TPW)])


def kernel(x, W, b):
    scores = _scores_tc(x, W, b.reshape(1, NE))
    rw, mk = _route_sc(scores)
    return rw, mk


# speedup vs baseline: 1.9196x; 1.9196x over previous
import functools

import jax
import jax.numpy as jnp
from jax import lax
from jax.experimental import pallas as pl
from jax.experimental.pallas import tpu as pltpu
from jax.experimental.pallas import tpu_sc as plsc

EMBED = 2048
NE = 16
NTOK = 16384
ACTIVE = 2

NC, NS, L = 2, 16, 16
NW = NC * NS
TPW = NTOK // NW
GROUPS = TPW // L

ROW_BLK = 1024


def _score_body(x_ref, w_ref, b_ref, o_ref):
    acc = jnp.dot(x_ref[...], w_ref[...], preferred_element_type=jnp.float32)
    o_ref[...] = jnp.maximum(acc + b_ref[...], 0.0)


def _scores_tc(x, W, b2):
    return pl.pallas_call(
        _score_body,
        grid=(NTOK // ROW_BLK,),
        in_specs=[
            pl.BlockSpec((ROW_BLK, EMBED), lambda i: (i, 0)),
            pl.BlockSpec((EMBED, NE), lambda i: (0, 0)),
            pl.BlockSpec((1, NE), lambda i: (0, 0)),
        ],
        out_specs=pl.BlockSpec((ROW_BLK, NE), lambda i: (i, 0)),
        out_shape=jax.ShapeDtypeStruct((NTOK, NE), jnp.float32),
    )(x, W, b2)


def _route_body(scores_hbm, rw_hbm, mk_hbm, s_v, rw_v, mk_v):
    wid = lax.axis_index("s") * NC + lax.axis_index("c")
    base = wid * (TPW * NE)
    pltpu.sync_copy(scores_hbm.at[pl.ds(base, TPW * NE)], s_v)

    zeros = jnp.zeros((L,), jnp.float32)
    ones = jnp.ones((L,), jnp.float32)

    def group(g, carry):
        row_starts = g * (L * NE) + lax.iota(jnp.int32, L) * NE
        m1 = jnp.full((L,), -jnp.inf, jnp.float32)
        m2 = jnp.full((L,), -jnp.inf, jnp.float32)
        i1 = jnp.zeros((L,), jnp.int32)
        i2 = jnp.zeros((L,), jnp.int32)
        for e in range(NE):
            v = plsc.load_gather(s_v, [row_starts + e])
            gt1 = v > m1
            gt2 = jnp.logical_and(jnp.logical_not(gt1), v > m2)
            i2 = jnp.where(gt1, i1, jnp.where(gt2, e, i2))
            m2 = jnp.where(gt1, m1, jnp.where(gt2, v, m2))
            i1 = jnp.where(gt1, e, i1)
            m1 = jnp.where(gt1, v, m1)
        t = jnp.exp(m2 - m1)
        den = 1.0 + t
        w1 = 1.0 / den
        w2 = t / den
        for r in range(L):
            start = g * (L * NE) + r * NE
            rw_v[pl.ds(start, NE)] = zeros
            mk_v[pl.ds(start, NE)] = zeros
        plsc.store_scatter(mk_v, [row_starts + i1], ones)
        plsc.store_scatter(mk_v, [row_starts + i2], ones)
        plsc.store_scatter(rw_v, [row_starts + i1], w1)
        plsc.store_scatter(rw_v, [row_starts + i2], w2)
        return carry

    lax.fori_loop(0, GROUPS, group, 0)
    pltpu.sync_copy(rw_v, rw_hbm.at[pl.ds(base, TPW * NE)])
    pltpu.sync_copy(mk_v, mk_hbm.at[pl.ds(base, TPW * NE)])


@functools.lru_cache(maxsize=1)
def _build_route_sc():
    return pl.kernel(
        _route_body,
        out_type=(
            jax.ShapeDtypeStruct((NTOK * NE,), jnp.float32),
            jax.ShapeDtypeStruct((NTOK * NE,), jnp.float32),
        ),
        mesh=plsc.VectorSubcoreMesh(core_axis_name="c", subcore_axis_name="s"),
        scratch_types=[
            pltpu.VMEM((TPW * NE,), jnp.float32),
            pltpu.VMEM((TPW * NE,), jnp.float32),
            pltpu.VMEM((TPW * NE,), jnp.float32),
        ],
        compiler_params=pltpu.CompilerParams(needs_layout_passes=False),
    )


def kernel(x, W, b):
    scores = _scores_tc(x, W, b.reshape(1, NE))
    rw, mk = _build_route_sc()(scores.reshape(NTOK * NE))
    return rw.reshape(NTOK, NE), mk.reshape(NTOK, NE)
